# baseline (speedup 1.0000x reference)
import functools

import jax
import jax.numpy as jnp
from jax import lax
from jax.experimental import pallas as pl
from jax.experimental.pallas import tpu as pltpu
from jax.experimental.pallas import tpu_sc as plsc

VOCAB = 1000000
D = 64
B = 4096
S = 200
PADW = 128
NC, NS = 2, 16
NW = NC * NS
BPW = B // NW
POS_FLAT = S * D
LANES = 16

_mesh = plsc.VectorSubcoreMesh(core_axis_name="c", subcore_axis_name="s")


@functools.partial(
    pl.kernel,
    mesh=_mesh,
    out_type=jax.ShapeDtypeStruct((S, D, B), jnp.float32),
    scratch_types=[
        pltpu.VMEM((BPW * S,), jnp.int32),
        pltpu.VMEM((S, BPW), jnp.int32),
        pltpu.VMEM((BPW, PADW), jnp.float32),
        pltpu.VMEM((BPW, PADW), jnp.float32),
        pltpu.VMEM((D, BPW), jnp.float32),
        pltpu.VMEM((D, BPW), jnp.float32),
        pltpu.VMEM((POS_FLAT,), jnp.float32),
        pltpu.SemaphoreType.DMA,
        pltpu.SemaphoreType.DMA,
        pltpu.SemaphoreType.DMA,
        pltpu.SemaphoreType.DMA,
    ],
    compiler_params=pltpu.CompilerParams(
        use_tc_tiling_on_sc=True, needs_layout_passes=False),
)
def _embed_sc(ids_hbm, tab_hbm, pos_hbm, out_hbm,
              idf, idt, g0, g1, t0, t1, posv, sg0, sg1, so0, so1):
    wid = lax.axis_index("s") * NC + lax.axis_index("c")
    b0 = wid * BPW

    pltpu.sync_copy(ids_hbm.at[pl.ds(b0 * S, BPW * S)], idf)
    pltpu.sync_copy(pos_hbm.at[pl.ds(0, POS_FLAT)], posv)

    lanes = lax.iota(jnp.int32, LANES)

    def tbody(s, c):
        for jb in range(BPW // LANES):
            src = (jb * LANES + lanes) * S + s
            idt[s, pl.ds(jb * LANES, LANES)] = plsc.load_gather(idf, [src])
        return c
    lax.fori_loop(0, S, tbody, 0)

    def fire(s, gbuf, sem):
        pltpu.async_copy(tab_hbm.at[idt.at[s]], gbuf, sem)

    def drain(gbuf, sem):
        pltpu.make_async_copy(tab_hbm.at[pl.ds(0, BPW)], gbuf, sem).wait()

    def process(s, gbuf, tbuf):
        def dbody(d, c):
            psplat = plsc.load_gather(posv, [jnp.broadcast_to(s * D + d, (LANES,))])
            cols = jnp.broadcast_to(d, (LANES,))
            for jb in range(BPW // LANES):
                rows = jb * LANES + lanes
                val = plsc.load_gather(gbuf, [rows, cols]) + psplat
                tbuf[d, pl.ds(jb * LANES, LANES)] = val
            return c
        lax.fori_loop(0, D, dbody, 0)

    def put(s, tbuf, sem):
        pltpu.async_copy(tbuf, out_hbm.at[s, :, pl.ds(b0, BPW)], sem)

    def wait_put(tbuf, sem):
        pltpu.make_async_copy(tbuf, out_hbm.at[0, :, pl.ds(b0, BPW)], sem).wait()

    fire(0, g0, sg0)

    def pair(i, c):
        s = 2 * i
        fire(s + 1, g1, sg1)
        drain(g0, sg0)

        @pl.when(i > 0)
        def _():
            wait_put(t0, so0)

        process(s, g0, t0)
        put(s, t0, so0)

        @pl.when(s + 2 < S)
        def _():
            fire(s + 2, g0, sg0)

        drain(g1, sg1)

        @pl.when(i > 0)
        def _():
            wait_put(t1, so1)

        process(s + 1, g1, t1)
        put(s + 1, t1, so1)
        return c

    lax.fori_loop(0, S // 2, pair, 0)
    wait_put(t0, so0)
    wait_put(t1, so1)


def kernel(input_ids, token_table, pos_table):
    ids1 = input_ids.reshape(B * S).astype(jnp.int32)
    tpad = jnp.pad(token_table, ((0, 0), (0, PADW - D)))
    pos1 = pos_table.reshape(-1)
    out = _embed_sc(ids1, tpad, pos1)
    return out.transpose(2, 0, 1)
